# baseline (speedup 1.0000x reference)
import jax
import jax.numpy as jnp
from jax.experimental import pallas as pl
from jax.experimental.pallas import tpu as pltpu

H = 1024
I = 512
E = 8
BT = 256


def _moe_body(x_ref, rw_ref, shg_ref, shu_ref, shd_ref,
              eg_ref, eu_ref, ed_ref, out_ref, mw_ref):
    e = pl.program_id(1)
    x = x_ref[...]

    @pl.when(e == 0)
    def _first():
        xb = x.astype(jnp.bfloat16)
        g = jnp.dot(xb, shg_ref[...].astype(jnp.bfloat16),
                    preferred_element_type=jnp.float32)
        u = jnp.dot(xb, shu_ref[...].astype(jnp.bfloat16),
                    preferred_element_type=jnp.float32)
        h = (jax.nn.sigmoid(g) * u).astype(jnp.bfloat16)
        shared = jnp.dot(h, shd_ref[...].astype(jnp.bfloat16),
                         preferred_element_type=jnp.float32)
        out_ref[...] = x + shared
        logits = jnp.dot(x, rw_ref[...].T, preferred_element_type=jnp.float32)
        logits = logits - jnp.max(logits, axis=1, keepdims=True)
        ex = jnp.exp(logits)
        probs = ex / jnp.sum(ex, axis=1, keepdims=True)
        idx8 = jax.lax.broadcasted_iota(jnp.int32, (BT, E), 1)
        m1 = jnp.max(probs, axis=1, keepdims=True)
        a1 = jnp.min(jnp.where(probs == m1, idx8, E), axis=1, keepdims=True)
        mask1 = idx8 == a1
        probs2 = jnp.where(mask1, -jnp.inf, probs)
        m2 = jnp.max(probs2, axis=1, keepdims=True)
        a2 = jnp.min(jnp.where(probs2 == m2, idx8, E), axis=1, keepdims=True)
        mask2 = idx8 == a2
        mw_ref[...] = jnp.where(mask1, m1, 0.0) + jnp.where(mask2, m2, 0.0)

    idx8 = jax.lax.broadcasted_iota(jnp.int32, (BT, E), 1)
    w_col = jnp.sum(jnp.where(idx8 == e, mw_ref[...], 0.0), axis=1,
                    keepdims=True)
    xb = x.astype(jnp.bfloat16)
    g = jnp.dot(xb, eg_ref[0].astype(jnp.bfloat16),
                preferred_element_type=jnp.float32)
    u = jnp.dot(xb, eu_ref[0].astype(jnp.bfloat16),
                preferred_element_type=jnp.float32)
    h = (jax.nn.sigmoid(g) * u).astype(jnp.bfloat16)
    y = jnp.dot(h, ed_ref[0].astype(jnp.bfloat16),
                preferred_element_type=jnp.float32)
    out_ref[...] += y * w_col


def kernel(x, router_w, sh_gate, sh_up, sh_down, exp_gate, exp_up, exp_down):
    Bb, Ss, Hh = x.shape
    flat = x.reshape(-1, Hh)
    T = flat.shape[0]
    nt = T // BT

    out = pl.pallas_call(
        _moe_body,
        grid=(nt, E),
        in_specs=[
            pl.BlockSpec((BT, H), lambda t, e: (t, 0)),
            pl.BlockSpec((E, H), lambda t, e: (0, 0)),
            pl.BlockSpec((H, I), lambda t, e: (0, 0)),
            pl.BlockSpec((H, I), lambda t, e: (0, 0)),
            pl.BlockSpec((I, H), lambda t, e: (0, 0)),
            pl.BlockSpec((1, H, I), lambda t, e: (e, 0, 0)),
            pl.BlockSpec((1, H, I), lambda t, e: (e, 0, 0)),
            pl.BlockSpec((1, I, H), lambda t, e: (e, 0, 0)),
        ],
        out_specs=pl.BlockSpec((BT, H), lambda t, e: (t, 0)),
        out_shape=jax.ShapeDtypeStruct((T, H), jnp.float32),
        scratch_shapes=[pltpu.VMEM((BT, E), jnp.float32)],
    )(flat, router_w, sh_gate, sh_up, sh_down, exp_gate, exp_up, exp_down)

    return out.reshape(Bb, Ss, Hh)


# speedup vs baseline: 1.9177x; 1.9177x over previous
import functools
import jax
import jax.numpy as jnp
from jax.experimental import pallas as pl
from jax.experimental.pallas import tpu as pltpu

H = 1024
I = 512
E = 8

_dot = functools.partial(jnp.dot, preferred_element_type=jnp.float32,
                         precision=jax.lax.Precision.DEFAULT)


def _moe_body(x_ref, rw_ref, shg_ref, shu_ref, shd_ref,
              eg_ref, eu_ref, ed_ref, out_ref, mw_ref):
    e = pl.program_id(0)
    x = x_ref[...]
    T = x.shape[0]

    @pl.when(e == 0)
    def _first():
        g = _dot(x, shg_ref[...])
        u = _dot(x, shu_ref[...])
        h = jax.nn.sigmoid(g) * u
        shared = _dot(h, shd_ref[...])
        out_ref[...] = x + shared
        logits = jax.lax.dot_general(
            x, rw_ref[...], (((1,), (1,)), ((), ())),
            preferred_element_type=jnp.float32)
        logits = logits - jnp.max(logits, axis=1, keepdims=True)
        ex = jnp.exp(logits)
        probs = ex / jnp.sum(ex, axis=1, keepdims=True)
        idx8 = jax.lax.broadcasted_iota(jnp.int32, (T, E), 1)
        m1 = jnp.max(probs, axis=1, keepdims=True)
        a1 = jnp.min(jnp.where(probs == m1, idx8, E), axis=1, keepdims=True)
        mask1 = idx8 == a1
        probs2 = jnp.where(mask1, -jnp.inf, probs)
        m2 = jnp.max(probs2, axis=1, keepdims=True)
        a2 = jnp.min(jnp.where(probs2 == m2, idx8, E), axis=1, keepdims=True)
        mask2 = idx8 == a2
        mw_ref[...] = jnp.where(mask1, m1, 0.0) + jnp.where(mask2, m2, 0.0)

    idx8 = jax.lax.broadcasted_iota(jnp.int32, (T, E), 1)
    w_col = jnp.sum(jnp.where(idx8 == e, mw_ref[...], 0.0), axis=1,
                    keepdims=True)
    g = _dot(x, eg_ref[0])
    u = _dot(x, eu_ref[0])
    h = jax.nn.sigmoid(g) * u
    y = _dot(h, ed_ref[0])
    out_ref[...] += y * w_col


def kernel(x, router_w, sh_gate, sh_up, sh_down, exp_gate, exp_up, exp_down):
    Bb, Ss, Hh = x.shape
    flat = x.reshape(-1, Hh)
    T = flat.shape[0]

    out = pl.pallas_call(
        _moe_body,
        grid=(E,),
        in_specs=[
            pl.BlockSpec((T, H), lambda e: (0, 0)),
            pl.BlockSpec((E, H), lambda e: (0, 0)),
            pl.BlockSpec((H, I), lambda e: (0, 0)),
            pl.BlockSpec((H, I), lambda e: (0, 0)),
            pl.BlockSpec((I, H), lambda e: (0, 0)),
            pl.BlockSpec((1, H, I), lambda e: (e, 0, 0)),
            pl.BlockSpec((1, H, I), lambda e: (e, 0, 0)),
            pl.BlockSpec((1, I, H), lambda e: (e, 0, 0)),
        ],
        out_specs=pl.BlockSpec((T, H), lambda e: (0, 0)),
        out_shape=jax.ShapeDtypeStruct((T, H), jnp.float32),
        scratch_shapes=[pltpu.VMEM((T, E), jnp.float32)],
    )(flat, router_w, sh_gate, sh_up, sh_down, exp_gate, exp_up, exp_down)

    return out.reshape(Bb, Ss, Hh)
